# Initial kernel scaffold; baseline (speedup 1.0000x reference)
import jax
import jax.numpy as jnp
from jax.experimental import pallas as pl


def kernel(x, edge_index, W1, a_src1, a_dst1, b1, W2, a_src2, a_dst2, b2, cand_W, cand_b):
    raise NotImplementedError("write your pallas kernel here")


# speedup vs baseline: 4.5985x; 4.5985x over previous
import functools

import jax
import jax.numpy as jnp
from jax import lax
from jax.experimental import pallas as pl
from jax.experimental.pallas import tpu as pltpu
from jax.experimental.pallas import tpu_sc as plsc

N = 10000
E = 320000
D = 128
H = 402
NUM_CAND = 100

HP = 416
NV = HP // 16
NPADT = 10016
CHUNK = 2000
NCH = E // CHUNK
MB = 2048
FLUSH = MB - 16


def _sread_f(ref16, j):
    v = ref16[pl.ds(0, 16)]
    return jnp.sum(jnp.where(lax.iota(jnp.int32, 16) == j, v, 0.0))


def _sread_i(ref16, j):
    v = ref16[pl.ds(0, 16)]
    return jnp.sum(jnp.where(lax.iota(jnp.int32, 16) == j, v, 0))


def _make_sc_layer(rng, npass, base, out_rows):
    mesh = plsc.VectorSubcoreMesh(core_axis_name="c", subcore_axis_name="s")
    den_pad = max(16, ((rng + 15) // 16) * 16)

    @functools.partial(
        pl.kernel,
        out_type=jax.ShapeDtypeStruct((out_rows, HP), jnp.float32),
        mesh=mesh,
        compiler_params=pltpu.CompilerParams(needs_layout_passes=False, use_tc_tiling_on_sc=False),
        scratch_types=[
            pltpu.VMEM((NPADT,), jnp.float32),
            pltpu.VMEM((NPADT,), jnp.float32),
            pltpu.VMEM((CHUNK,), jnp.int32),
            pltpu.VMEM((CHUNK,), jnp.int32),
            pltpu.VMEM((MB,), jnp.int32),
            pltpu.VMEM((MB,), jnp.int32),
            pltpu.VMEM((16,), jnp.float32),
            pltpu.VMEM((16,), jnp.int32),
            pltpu.VMEM((16, HP), jnp.float32),
            pltpu.VMEM((rng, HP), jnp.float32),
            pltpu.VMEM((den_pad,), jnp.float32),
            pltpu.SemaphoreType.DMA,
        ],
    )
    def f(src_h, dst_h, as_h, ad_h, h_h, out_h,
          as_t, ad_t, src_c, dst_c, msrc, mdst, pbuf, dbuf, rows, acc, den,
          sem):
        wid = lax.axis_index("s") * 2 + lax.axis_index("c")
        pltpu.sync_copy(as_h, as_t)
        pltpu.sync_copy(ad_h, ad_t)
        zero16 = jnp.zeros((16,), jnp.float32)
        iota = lax.iota(jnp.int32, 16)

        def process(mc, lo):
            nb = (mc + 15) // 16

            def batch(k, _):
                s16 = msrc[pl.ds(k * 16, 16)]
                d16 = mdst[pl.ds(k * 16, 16)]
                valid = (iota + k * 16) < mc
                s16 = jnp.where(valid, s16, 0)
                d16 = jnp.where(valid, d16, lo)
                a1 = plsc.load_gather(as_t, [s16])
                a2 = plsc.load_gather(ad_t, [d16])
                e = a1 + a2
                e = jnp.where(e > 0, e, 0.2 * e)
                p = jnp.where(valid, jnp.exp(e), 0.0)
                pbuf[pl.ds(0, 16)] = p
                dbuf[pl.ds(0, 16)] = d16
                pltpu.async_copy(h_h.at[s16], rows, sem).wait()

                def edge(j, _):
                    dl = _sread_i(dbuf, j) - lo
                    pj = _sread_f(pbuf, j)
                    dw = (dl // 16) * 16
                    io2 = iota + dw
                    den[pl.ds(dw, 16)] = den[pl.ds(dw, 16)] + jnp.where(
                        io2 == dl, pj, 0.0)
                    pv = jnp.full((16,), pj, jnp.float32)
                    for c in range(NV):
                        acc[dl, pl.ds(c * 16, 16)] = (
                            acc[dl, pl.ds(c * 16, 16)]
                            + pv * rows[j, pl.ds(c * 16, 16)])
                    return 0

                lax.fori_loop(0, 16, edge, 0)
                return 0

            lax.fori_loop(0, nb, batch, 0)

        def do_pass(p, _):
            lo = base + (wid * npass + p) * rng

            def zrow(r, _):
                for c in range(NV):
                    acc[r, pl.ds(c * 16, 16)] = zero16
                return 0
            lax.fori_loop(0, rng, zrow, 0)
            for i in range(den_pad // 16):
                den[pl.ds(i * 16, 16)] = zero16

            def chunk_loop(ci, mc):
                pltpu.sync_copy(src_h.at[pl.ds(ci * CHUNK, CHUNK)], src_c)
                pltpu.sync_copy(dst_h.at[pl.ds(ci * CHUNK, CHUNK)], dst_c)

                def vreg_loop(v, mc):
                    d = dst_c[pl.ds(v * 16, 16)]
                    m = (d >= lo) & (d < lo + rng)
                    cnt = jnp.sum(m.astype(jnp.int32))

                    def append(mc):
                        s = src_c[pl.ds(v * 16, 16)]
                        plsc.store_compressed(msrc.at[pl.ds(mc, 16)], s, mask=m)
                        plsc.store_compressed(mdst.at[pl.ds(mc, 16)], d, mask=m)
                        return mc + cnt

                    mc = lax.cond(cnt > 0, append, lambda mc: mc, mc)

                    def flush(mc):
                        process(mc, lo)
                        return 0

                    return lax.cond(mc >= FLUSH, flush, lambda mc: mc, mc)

                return lax.fori_loop(0, CHUNK // 16, vreg_loop, mc)

            mc = lax.fori_loop(0, NCH, chunk_loop, 0)
            process(mc, lo)

            def nrow(r, _):
                dv = den[pl.ds((r // 16) * 16, 16)]
                dr = jnp.sum(jnp.where(iota + (r // 16) * 16 == r, dv, 0.0))
                rv = 1.0 / (jnp.full((16,), dr, jnp.float32) + 1e-16)
                for c in range(NV):
                    acc[r, pl.ds(c * 16, 16)] = acc[r, pl.ds(c * 16, 16)] * rv
                return 0
            lax.fori_loop(0, rng, nrow, 0)
            pltpu.sync_copy(acc, out_h.at[pl.ds(lo - base, rng)])
            return 0

        lax.fori_loop(0, npass, do_pass, 0)

    return f


def _tc_matmul(xp, wp, bp=None, relu=False, block_m=1000):
    m, k = xp.shape
    kout = wp.shape[1]

    def body(x_ref, w_ref, b_ref, o_ref):
        xv = x_ref[...]
        if b_ref is not None:
            xv = xv + b_ref[...]
        if relu:
            xv = jnp.maximum(xv, 0.0)
        o_ref[...] = jnp.dot(xv, w_ref[...], preferred_element_type=jnp.float32,
                             precision=jax.lax.Precision.HIGHEST)

    grid = (m // block_m,)
    in_specs = [
        pl.BlockSpec((block_m, k), lambda i: (i, 0)),
        pl.BlockSpec((k, kout), lambda i: (0, 0)),
    ]
    args = [xp, wp]
    if bp is not None:
        in_specs.append(pl.BlockSpec((1, k), lambda i: (0, 0)))
        args.append(bp.reshape(1, k))
        f = lambda x_ref, w_ref, b_ref, o_ref: body(x_ref, w_ref, b_ref, o_ref)
    else:
        f = lambda x_ref, w_ref, o_ref: body(x_ref, w_ref, None, o_ref)

    return pl.pallas_call(
        f,
        grid=grid,
        in_specs=in_specs,
        out_specs=pl.BlockSpec((block_m, kout), lambda i: (i, 0)),
        out_shape=jax.ShapeDtypeStruct((m, kout), jnp.float32),
    )(*args)


def kernel(x, edge_index, W1, a_src1, a_dst1, b1, W2, a_src2, a_dst2, b2,
           cand_W, cand_b):
    f32 = jnp.float32
    src = edge_index[0].astype(jnp.int32)
    dst = edge_index[1].astype(jnp.int32)

    w1cat = jnp.zeros((D, 512), f32)
    w1cat = w1cat.at[:, :H].set(W1)
    w1cat = w1cat.at[:, HP].set(W1 @ a_src1)
    w1cat = w1cat.at[:, HP + 1].set(W1 @ a_dst1)

    w2cat = jnp.zeros((512, 512), f32)
    w2cat = w2cat.at[:H, :H].set(W2)
    w2cat = w2cat.at[:H, HP].set(W2 @ a_src2)
    w2cat = w2cat.at[:H, HP + 1].set(W2 @ a_dst2)

    b1p = jnp.zeros((512,), f32).at[:H].set(b1)
    b2p = jnp.zeros((512,), f32).at[:H].set(b2)
    cwp = jnp.zeros((512, 128), f32).at[:H, 0].set(cand_W[:, 0])

    out1 = _tc_matmul(x, w1cat)
    h1 = out1[:, :HP]
    as1 = jnp.zeros((NPADT,), f32).at[:N].set(out1[:, HP])
    ad1 = jnp.zeros((NPADT,), f32).at[:N].set(out1[:, HP + 1])

    sc1 = _make_sc_layer(rng=160, npass=2, base=0, out_rows=10240)
    agg1 = sc1(src, dst, as1, ad1, h1)[:N]

    agg1p = jnp.zeros((N, 512), f32).at[:, :HP].set(agg1)
    out2 = _tc_matmul(agg1p, w2cat, bp=b1p, relu=True)
    h2 = out2[:, :HP]
    as2 = jnp.zeros((NPADT,), f32).at[:N].set(out2[:, HP])
    ad2 = jnp.zeros((NPADT,), f32).at[:N].set(out2[:, HP + 1])

    sc2 = _make_sc_layer(rng=8, npass=1, base=N - NUM_CAND, out_rows=256)
    agg2 = sc2(src, dst, as2, ad2, h2)[:NUM_CAND]

    agg2p = jnp.zeros((104, 512), f32).at[:NUM_CAND, :HP].set(agg2)
    res = _tc_matmul(agg2p, cwp, bp=b2p, relu=True, block_m=104)
    return res[:NUM_CAND, 0] + cand_b[0]


# speedup vs baseline: 15.8794x; 3.4532x over previous
import functools

import jax
import jax.numpy as jnp
from jax import lax
from jax.experimental import pallas as pl
from jax.experimental.pallas import tpu as pltpu
from jax.experimental.pallas import tpu_sc as plsc

N = 10000
E = 320000
D = 128
H = 402
NUM_CAND = 100

HP = 416
NV = HP // 16
NPADT = 10016
SHARD = E // 16
CHUNK = 2000
NCH = SHARD // CHUNK
MB = 2048
FLUSH = MB - 16


def _exp_f32(x):
    y = x * 1.4426950408889634
    t = y + 12582912.0
    yi = t - 12582912.0
    f = (y - yi) * 0.6931471805599453
    poly = 1.0 + f * (1.0 + f * (0.5 + f * (1.6666666666666666e-1 + f * (
        4.1666666666666664e-2 + f * (8.333333333333333e-3 + f * 1.3888888888888889e-3)))))
    scale = plsc.bitcast(
        jax.lax.shift_left(yi.astype(jnp.int32) + 127, jnp.full((16,), 23, jnp.int32)),
        jnp.float32)
    return scale * poly


def _make_sc_layer(sc_rows, base, sub_rows, norm_chunks):
    mesh = plsc.VectorSubcoreMesh(core_axis_name="c", subcore_axis_name="s")
    nbuf_rows = max(sz for _, sz in norm_chunks)
    den_pad = ((sc_rows + 15) // 16) * 16

    @functools.partial(
        pl.kernel,
        out_type=jax.ShapeDtypeStruct((2 * sc_rows, HP), jnp.float32),
        mesh=mesh,
        compiler_params=pltpu.CompilerParams(needs_layout_passes=False,
                                             use_tc_tiling_on_sc=False),
        scratch_types=[
            pltpu.VMEM((NPADT,), jnp.float32),
            pltpu.VMEM((NPADT,), jnp.float32),
            pltpu.VMEM((CHUNK,), jnp.int32),
            pltpu.VMEM((CHUNK,), jnp.int32),
            pltpu.VMEM((MB,), jnp.int32),
            pltpu.VMEM((MB,), jnp.int32),
            pltpu.VMEM((16,), jnp.float32),
            pltpu.VMEM((16,), jnp.int32),
            pltpu.VMEM((32,), jnp.int32),
            pltpu.VMEM((16,), jnp.int32),
            pltpu.VMEM((16, HP), jnp.float32),
            pltpu.VMEM((nbuf_rows, HP), jnp.float32),
            pltpu.VMEM_SHARED((sc_rows, HP), jnp.float32),
            pltpu.SemaphoreType.DMA,
        ],
    )
    def f(src_h, dst_h, as_h, ad_h, h_h, out_h,
          as_t, ad_t, src_c, dst_c, msrc, mdst, pbuf, dlbuf, dpad, fbuf,
          rows, nbuf, acc_sh, sem):
        cid = lax.axis_index("c")
        sid = lax.axis_index("s")
        wid = sid * 2 + cid
        lo = base + cid * sc_rows
        pltpu.sync_copy(as_h, as_t)
        pltpu.sync_copy(ad_h, ad_t)
        zero16 = jnp.zeros((16,), jnp.float32)
        iota = lax.iota(jnp.int32, 16)

        for c in range(NV):
            nbuf[0, pl.ds(c * 16, 16)] = zero16

        def zrow(r, _):
            pltpu.sync_copy(nbuf.at[pl.ds(0, 1)],
                            acc_sh.at[pl.ds(sid * sub_rows + r, 1)])
            return 0
        lax.fori_loop(0, sub_rows, zrow, 0)

        plsc.subcore_barrier()

        def process(mc):
            nb = (mc + 15) // 16

            def batch(k, _):
                s16 = msrc[pl.ds(k * 16, 16)]
                d16 = mdst[pl.ds(k * 16, 16)]
                valid = (iota + k * 16) < mc
                s16 = jnp.where(valid, s16, 0)
                d16 = jnp.where(valid, d16, lo)
                a1 = plsc.load_gather(as_t, [s16])
                a2 = plsc.load_gather(ad_t, [d16])
                e = a1 + a2
                e = jnp.where(e > 0, e, 0.2 * e)
                p = jnp.where(valid, _exp_f32(e), 0.0)
                pbuf[pl.ds(0, 16)] = p
                dl = d16 - lo
                dlbuf[pl.ds(0, 16)] = dl
                dpad[pl.ds(0, 16)] = jnp.full((16,), -1, jnp.int32)
                dpad[pl.ds(16, 16)] = dl
                fidx = iota
                for r in range(1, 16):
                    prev = dpad[pl.ds(16 - r, 16)]
                    fidx = jnp.where(dl == prev, iota - r, fidx)
                fbuf[pl.ds(0, 16)] = fidx
                cntd = jnp.sum((fidx != iota).astype(jnp.int32))
                pltpu.async_copy(h_h.at[s16], rows, sem).wait()

                def scale(j, _):
                    pv = plsc.load_gather(
                        pbuf, [jnp.full((16,), j, jnp.int32)])
                    last = rows[j, pl.ds(400, 16)]
                    last = jnp.where(iota == H - 400, 1.0, last)
                    rows[j, pl.ds(400, 16)] = last * pv
                    for c in range(NV - 1):
                        rows[j, pl.ds(c * 16, 16)] = (
                            rows[j, pl.ds(c * 16, 16)] * pv)
                    return 0
                lax.fori_loop(0, 16, scale, 0)

                @pl.when(cntd > 0)
                def _():
                    def dedup(j, _):
                        fv = fbuf[pl.ds(0, 16)]
                        fj = jnp.sum(jnp.where(iota == j, fv, 0))

                        @pl.when(fj != j)
                        def _():
                            for c in range(NV):
                                rows[fj, pl.ds(c * 16, 16)] = (
                                    rows[fj, pl.ds(c * 16, 16)]
                                    + rows[j, pl.ds(c * 16, 16)])
                            for c in range(NV):
                                rows[j, pl.ds(c * 16, 16)] = zero16
                        return 0

                    lax.fori_loop(0, 16, dedup, 0)

                pltpu.sync_copy(rows, acc_sh.at[dlbuf], add=True)
                return 0

            lax.fori_loop(0, nb, batch, 0)

        def chunk_loop(ci, mc):
            ebase = sid * SHARD + ci * CHUNK
            pltpu.sync_copy(src_h.at[pl.ds(ebase, CHUNK)], src_c)
            pltpu.sync_copy(dst_h.at[pl.ds(ebase, CHUNK)], dst_c)

            def vreg_loop(v, mc):
                d = dst_c[pl.ds(v * 16, 16)]
                m = (d >= lo) & (d < lo + sc_rows)
                cnt = jnp.sum(m.astype(jnp.int32))

                def append(mc):
                    s = src_c[pl.ds(v * 16, 16)]
                    plsc.store_compressed(msrc.at[pl.ds(mc, 16)], s, mask=m)
                    plsc.store_compressed(mdst.at[pl.ds(mc, 16)], d, mask=m)
                    return mc + cnt

                mc = lax.cond(cnt > 0, append, lambda mc: mc, mc)

                def flush(mc):
                    process(mc)
                    return 0

                return lax.cond(mc >= FLUSH, flush, lambda mc: mc, mc)

            return lax.fori_loop(0, CHUNK // 16, vreg_loop, mc)

        mc = lax.fori_loop(0, NCH, chunk_loop, 0)
        process(mc)

        plsc.subcore_barrier()

        rbase = sid * sub_rows

        for off, size in norm_chunks:
            nb_ref = nbuf.at[pl.ds(0, size)]
            pltpu.sync_copy(acc_sh.at[pl.ds(rbase + off, size)], nb_ref)

            def nrow(r, _, off=off):
                dv = nbuf[r, pl.ds(400, 16)]
                dr = jnp.sum(jnp.where(iota == H - 400, dv, 0.0))
                rv = 1.0 / (jnp.full((16,), dr, jnp.float32) + 1e-16)
                for c in range(NV):
                    nbuf[r, pl.ds(c * 16, 16)] = (
                        nbuf[r, pl.ds(c * 16, 16)] * rv)
                return 0
            lax.fori_loop(0, size, nrow, 0)
            pltpu.sync_copy(
                nb_ref, out_h.at[pl.ds(cid * sc_rows + rbase + off, size)])

    return f


def _tc_matmul(xp, wp, bp=None, relu=False, block_m=1000):
    m, k = xp.shape
    kout = wp.shape[1]

    def body(x_ref, w_ref, b_ref, o_ref):
        xv = x_ref[...]
        if b_ref is not None:
            xv = xv + b_ref[...]
        if relu:
            xv = jnp.maximum(xv, 0.0)
        o_ref[...] = jnp.dot(xv, w_ref[...], preferred_element_type=jnp.float32)

    grid = (m // block_m,)
    in_specs = [
        pl.BlockSpec((block_m, k), lambda i: (i, 0)),
        pl.BlockSpec((k, kout), lambda i: (0, 0)),
    ]
    args = [xp, wp]
    if bp is not None:
        in_specs.append(pl.BlockSpec((1, k), lambda i: (0, 0)))
        args.append(bp.reshape(1, k))
        f = lambda x_ref, w_ref, b_ref, o_ref: body(x_ref, w_ref, b_ref, o_ref)
    else:
        f = lambda x_ref, w_ref, o_ref: body(x_ref, w_ref, None, o_ref)

    return pl.pallas_call(
        f,
        grid=grid,
        in_specs=in_specs,
        out_specs=pl.BlockSpec((block_m, kout), lambda i: (i, 0)),
        out_shape=jax.ShapeDtypeStruct((m, kout), jnp.float32),
    )(*args)


def kernel(x, edge_index, W1, a_src1, a_dst1, b1, W2, a_src2, a_dst2, b2,
           cand_W, cand_b):
    f32 = jnp.float32
    src = edge_index[0].astype(jnp.int32)
    dst = edge_index[1].astype(jnp.int32)

    w1cat = jnp.zeros((D, 512), f32)
    w1cat = w1cat.at[:, :H].set(W1)
    w1cat = w1cat.at[:, HP].set(W1 @ a_src1)
    w1cat = w1cat.at[:, HP + 1].set(W1 @ a_dst1)

    w2cat = jnp.zeros((512, 512), f32)
    w2cat = w2cat.at[:H, :H].set(W2)
    w2cat = w2cat.at[:H, HP].set(W2 @ a_src2)
    w2cat = w2cat.at[:H, HP + 1].set(W2 @ a_dst2)

    b1p = jnp.zeros((512,), f32).at[:H].set(b1)
    b2p = jnp.zeros((512,), f32).at[:H].set(b2)
    cwp = jnp.zeros((512, 128), f32).at[:H, 0].set(cand_W[:, 0])

    out1 = _tc_matmul(x, w1cat)
    h1 = out1[:, :HP]
    as1 = jnp.zeros((NPADT,), f32).at[:N].set(out1[:, HP])
    ad1 = jnp.zeros((NPADT,), f32).at[:N].set(out1[:, HP + 1])

    nc1 = [(0, 64), (64, 64), (128, 29)]
    sc1a = _make_sc_layer(sc_rows=2512, base=0, sub_rows=157, norm_chunks=nc1)
    sc1b = _make_sc_layer(sc_rows=2512, base=5024, sub_rows=157,
                          norm_chunks=nc1)
    agg1 = jnp.concatenate([sc1a(src, dst, as1, ad1, h1),
                            sc1b(src, dst, as1, ad1, h1)], axis=0)[:N]

    agg1p = jnp.zeros((N, 512), f32).at[:, :HP].set(agg1)
    out2 = _tc_matmul(agg1p, w2cat, bp=b1p, relu=True)
    h2 = out2[:, :HP]
    as2 = jnp.zeros((NPADT,), f32).at[:N].set(out2[:, HP])
    ad2 = jnp.zeros((NPADT,), f32).at[:N].set(out2[:, HP + 1])

    sc2 = _make_sc_layer(sc_rows=64, base=N - NUM_CAND, sub_rows=4,
                         norm_chunks=[(0, 4)])
    agg2 = sc2(src, dst, as2, ad2, h2)[:NUM_CAND]

    agg2p = jnp.zeros((104, 512), f32).at[:NUM_CAND, :HP].set(agg2)
    res = _tc_matmul(agg2p, cwp, bp=b2p, relu=True, block_m=104)
    return res[:NUM_CAND, 0] + cand_b[0]
